# baseline (speedup 1.0000x reference)
import functools

import jax
import jax.numpy as jnp
from jax import lax
from jax.experimental import pallas as pl
from jax.experimental.pallas import tpu as pltpu
from jax.experimental.pallas import tpu_sc as plsc

B = 2
Q = 100
NPTS = 50000
M = 30
NUM_CLASSES = 20
NUM_POINTS = 12288
W_CE = 2.0
W_MASK = 5.0
W_DICE = 5.0

MPAD = 32
NC = 2
NS = 16
NW = NC * NS
PTS_PER_W = (B * NUM_POINTS) // NW
GCHUNK = 128
LANES = 16


def _pred_gather(pred_hbm, fidx_hbm, x_hbm, idx_v, rows_v, xbuf, sem):
    wid = lax.axis_index("s") * NC + lax.axis_index("c")
    base = wid * PTS_PER_W
    pltpu.sync_copy(fidx_hbm.at[pl.ds(base, PTS_PER_W)], idx_v)
    cps = []
    for c in range(PTS_PER_W // GCHUNK):
        cps.append(pltpu.async_copy(
            pred_hbm.at[idx_v.at[pl.ds(c * GCHUNK, GCHUNK)]],
            rows_v.at[pl.ds(c * GCHUNK, GCHUNK)], sem))
    for cp in cps:
        cp.wait()
    iota = lax.iota(jnp.int32, LANES)

    def body(i, carry):
        ridx = i * LANES + iota
        for m in range(MPAD):
            col = jnp.full((LANES,), m, jnp.int32)
            xbuf[m, pl.ds(i * LANES, LANES)] = plsc.load_gather(
                rows_v, [ridx, col])
        return carry

    lax.fori_loop(0, PTS_PER_W // LANES, body, 0)
    b = base // NUM_POINTS
    col0 = base % NUM_POINTS
    pltpu.sync_copy(xbuf, x_hbm.at[b, :, pl.ds(col0, PTS_PER_W)])


def _tgt_gather(tgt_hbm, idx_hbm, t_hbm, row_v, idxb_v, tbuf):
    wid = lax.axis_index("s") * NC + lax.axis_index("c")

    def do_pair(p):
        b = p // M
        m = p % M
        pltpu.sync_copy(tgt_hbm.at[p], row_v)
        pltpu.sync_copy(idx_hbm.at[b], idxb_v)

        def body(i, carry):
            iv = idxb_v[pl.ds(i * LANES, LANES)]
            tbuf[pl.ds(i * LANES, LANES)] = plsc.load_gather(
                row_v, [lax.shift_right_logical(iv, 4),
                        lax.bitwise_and(iv, 15)])
            return carry

        lax.fori_loop(0, NUM_POINTS // LANES, body, 0)
        pltpu.sync_copy(tbuf, t_hbm.at[b, m])

    do_pair(wid)
    p2 = wid + NW

    @pl.when(p2 < B * M)
    def _():
        do_pair(p2)


def _loss(x_ref, t_ref, logits_ref, tgtcls_ref, cw_ref, out_ref):
    x = x_ref[...]
    t = t_ref[...]
    mrow = lax.broadcasted_iota(jnp.int32, (B, MPAD), 1) < M
    mmask = lax.broadcasted_iota(jnp.int32, (B, MPAD, 1), 1) < M
    x = jnp.where(mmask, x, 0.0)
    t = jnp.where(mmask, t, 0.0)
    s = 1.0 / (1.0 + jnp.exp(-x))
    num = 2.0 * jnp.sum(s * t, axis=-1)
    den = jnp.sum(s, axis=-1) + jnp.sum(t, axis=-1)
    dice_terms = 1.0 - (num + 1.0) / (den + 1.0)
    inv_masks = 1.0 / float(B * M)
    loss_dice = jnp.sum(jnp.where(mrow, dice_terms, 0.0)) * inv_masks
    bce = jnp.maximum(x, 0.0) - x * t + jnp.log1p(jnp.exp(-jnp.abs(x)))
    bce_row = jnp.sum(bce, axis=-1) * (1.0 / NUM_POINTS)
    loss_mask = jnp.sum(jnp.where(mrow, bce_row, 0.0)) * inv_masks

    logits = logits_ref[...]
    zmax = jnp.max(logits, axis=-1, keepdims=True)
    lse = zmax + jnp.log(jnp.sum(jnp.exp(logits - zmax), axis=-1,
                                 keepdims=True))
    logp = logits - lse
    tgt_full = jnp.concatenate(
        [tgtcls_ref[...],
         jnp.full((B, Q - M), NUM_CLASSES, jnp.int32)], axis=1)
    oh = (lax.broadcasted_iota(jnp.int32, (B, Q, NUM_CLASSES + 1), 2)
          == tgt_full[:, :, None])
    nll = -jnp.sum(jnp.where(oh, logp, 0.0), axis=-1)
    w = jnp.sum(jnp.where(oh, cw_ref[...], 0.0), axis=-1)
    loss_ce = jnp.sum(nll * w) / jnp.sum(w)

    out_ref[...] = jnp.stack(
        [W_CE * loss_ce, W_MASK * loss_mask, W_DICE * loss_dice])


def kernel(pred_logits, pred_masks, target_masks, target_classes,
           sampled_idx, class_weights):
    pred_flat = pred_masks.reshape(B * NPTS, Q)
    tgt_flat = target_masks.reshape(B * M, NPTS // LANES, LANES)
    idx = sampled_idx.astype(jnp.int32)
    fidx = (idx + (jnp.arange(B, dtype=jnp.int32) * NPTS)[:, None]).reshape(-1)
    mesh = plsc.VectorSubcoreMesh(core_axis_name="c", subcore_axis_name="s")

    x_g = pl.kernel(
        _pred_gather,
        out_type=jax.ShapeDtypeStruct((B, MPAD, NUM_POINTS), jnp.float32),
        mesh=mesh,
        compiler_params=pltpu.CompilerParams(
            needs_layout_passes=False, use_tc_tiling_on_sc=False),
        scratch_types=[
            pltpu.VMEM((PTS_PER_W,), jnp.int32),
            pltpu.VMEM((PTS_PER_W, Q), jnp.float32),
            pltpu.VMEM((MPAD, PTS_PER_W), jnp.float32),
            pltpu.SemaphoreType.DMA,
        ],
    )(pred_flat, fidx)

    t_g = pl.kernel(
        _tgt_gather,
        out_type=jax.ShapeDtypeStruct((B, MPAD, NUM_POINTS), jnp.float32),
        mesh=mesh,
        compiler_params=pltpu.CompilerParams(
            needs_layout_passes=False, use_tc_tiling_on_sc=False),
        scratch_types=[
            pltpu.VMEM((NPTS // LANES, LANES), jnp.float32),
            pltpu.VMEM((NUM_POINTS,), jnp.int32),
            pltpu.VMEM((NUM_POINTS,), jnp.float32),
        ],
    )(tgt_flat, idx)

    cw3 = class_weights.reshape(1, 1, NUM_CLASSES + 1)
    return pl.pallas_call(
        _loss,
        out_shape=jax.ShapeDtypeStruct((3,), jnp.float32),
    )(x_g, t_g, pred_logits.astype(jnp.float32),
      target_classes.astype(jnp.int32), cw3)


# speedup vs baseline: 1.6976x; 1.6976x over previous
import functools

import jax
import jax.numpy as jnp
from jax import lax
from jax.experimental import pallas as pl
from jax.experimental.pallas import tpu as pltpu
from jax.experimental.pallas import tpu_sc as plsc

B = 2
Q = 100
NPTS = 50000
M = 30
NUM_CLASSES = 20
NUM_POINTS = 12288
W_CE = 2.0
W_MASK = 5.0
W_DICE = 5.0

MPAD = 32
NC = 2
NS = 16
NW = NC * NS
LANES = 16
CP = 2048
PITCH = 50048
UNROLL = 4

_SC_PARAMS = pltpu.CompilerParams(
    needs_layout_passes=False, use_tc_tiling_on_sc=False)


def _transpose(x_ref, o_ref):
    x = x_ref[0]
    o_ref[0] = jnp.transpose(lax.slice(x, (0, 0), (CP, MPAD)), (1, 0))


def _gather_one_row(row_v, idxb_v, tbuf):
    def body(i, carry):
        base = i * (LANES * UNROLL)
        for u in range(UNROLL):
            iv = idxb_v[pl.ds(base + u * LANES, LANES)]
            tbuf[pl.ds(base + u * LANES, LANES)] = plsc.load_gather(
                row_v, [iv])
        return carry

    lax.fori_loop(0, NUM_POINTS // (LANES * UNROLL), body, 0)


def _tgt_gather(tgt_hbm, idx_hbm, t_hbm, row_v, idxb_v, tbuf):
    wid = lax.axis_index("s") * NC + lax.axis_index("c")

    def do_task(p):
        b = p // M
        m = p % M
        pltpu.sync_copy(tgt_hbm.at[p], row_v)
        pltpu.sync_copy(idx_hbm.at[b], idxb_v)
        _gather_one_row(row_v, idxb_v, tbuf)
        pltpu.sync_copy(tbuf, t_hbm.at[b, m])

    do_task(wid)
    p2 = wid + NW

    @pl.when(p2 < B * M)
    def _():
        do_task(p2)


def _pred_gather(predt_hbm, idx_hbm, x_hbm, row_v, idxb_v, tbuf):
    wid = lax.axis_index("s") * NC + lax.axis_index("c")

    def do_task(p):
        b = p // MPAD
        m = p % MPAD
        pltpu.sync_copy(predt_hbm.at[pl.ds(p * PITCH, NPTS)], row_v)
        pltpu.sync_copy(idx_hbm.at[b], idxb_v)
        _gather_one_row(row_v, idxb_v, tbuf)
        pltpu.sync_copy(tbuf, x_hbm.at[b, m])

    do_task(wid)
    do_task(wid + NW)


def _loss(x_ref, t_ref, logits_ref, tgtcls_ref, cw_ref, out_ref):
    x = x_ref[...]
    t = t_ref[...]
    mrow = lax.broadcasted_iota(jnp.int32, (B, MPAD), 1) < M
    mmask = lax.broadcasted_iota(jnp.int32, (B, MPAD, 1), 1) < M
    x = jnp.where(mmask, x, 0.0)
    t = jnp.where(mmask, t, 0.0)
    s = 1.0 / (1.0 + jnp.exp(-x))
    num = 2.0 * jnp.sum(s * t, axis=-1)
    den = jnp.sum(s, axis=-1) + jnp.sum(t, axis=-1)
    dice_terms = 1.0 - (num + 1.0) / (den + 1.0)
    inv_masks = 1.0 / float(B * M)
    loss_dice = jnp.sum(jnp.where(mrow, dice_terms, 0.0)) * inv_masks
    bce = jnp.maximum(x, 0.0) - x * t + jnp.log1p(jnp.exp(-jnp.abs(x)))
    bce_row = jnp.sum(bce, axis=-1) * (1.0 / NUM_POINTS)
    loss_mask = jnp.sum(jnp.where(mrow, bce_row, 0.0)) * inv_masks

    logits = logits_ref[...]
    zmax = jnp.max(logits, axis=-1, keepdims=True)
    lse = zmax + jnp.log(jnp.sum(jnp.exp(logits - zmax), axis=-1,
                                 keepdims=True))
    logp = logits - lse
    tgt_full = jnp.concatenate(
        [tgtcls_ref[...],
         jnp.full((B, Q - M), NUM_CLASSES, jnp.int32)], axis=1)
    oh = (lax.broadcasted_iota(jnp.int32, (B, Q, NUM_CLASSES + 1), 2)
          == tgt_full[:, :, None])
    nll = -jnp.sum(jnp.where(oh, logp, 0.0), axis=-1)
    w = jnp.sum(jnp.where(oh, cw_ref[...], 0.0), axis=-1)
    loss_ce = jnp.sum(nll * w) / jnp.sum(w)

    out_ref[...] = jnp.stack(
        [W_CE * loss_ce, W_MASK * loss_mask, W_DICE * loss_dice])


def kernel(pred_logits, pred_masks, target_masks, target_classes,
           sampled_idx, class_weights):
    idx = sampled_idx.astype(jnp.int32)
    tgt2d = target_masks.reshape(B * M, NPTS)
    mesh = plsc.VectorSubcoreMesh(core_axis_name="c", subcore_axis_name="s")

    predt = pl.pallas_call(
        _transpose,
        grid=(B, 25),
        in_specs=[pl.BlockSpec((1, CP, Q), lambda b, j: (b, j, 0))],
        out_specs=pl.BlockSpec((1, MPAD, CP), lambda b, j: (b, 0, j)),
        out_shape=jax.ShapeDtypeStruct((B, MPAD, PITCH), jnp.float32),
    )(pred_masks)
    predt_flat = predt.reshape(B * MPAD * PITCH)

    sc_scratch = [
        pltpu.VMEM((NPTS,), jnp.float32),
        pltpu.VMEM((NUM_POINTS,), jnp.int32),
        pltpu.VMEM((NUM_POINTS,), jnp.float32),
    ]
    t_g = pl.kernel(
        _tgt_gather,
        out_type=jax.ShapeDtypeStruct((B, MPAD, NUM_POINTS), jnp.float32),
        mesh=mesh,
        compiler_params=_SC_PARAMS,
        scratch_types=sc_scratch,
    )(tgt2d, idx)

    x_g = pl.kernel(
        _pred_gather,
        out_type=jax.ShapeDtypeStruct((B, MPAD, NUM_POINTS), jnp.float32),
        mesh=mesh,
        compiler_params=_SC_PARAMS,
        scratch_types=sc_scratch,
    )(predt_flat, idx)

    cw3 = class_weights.reshape(1, 1, NUM_CLASSES + 1)
    return pl.pallas_call(
        _loss,
        out_shape=jax.ShapeDtypeStruct((3,), jnp.float32),
    )(x_g, t_g, pred_logits.astype(jnp.float32),
      target_classes.astype(jnp.int32), cw3)
